# baseline (speedup 1.0000x reference)
import functools

import jax
import jax.numpy as jnp
from jax import lax
from jax.experimental import pallas as pl
from jax.experimental.pallas import tpu as pltpu
from jax.experimental.pallas import tpu_sc as plsc

B = 4096
L = 200
D = 64
EPS = 1e-12

_info = plsc.get_sparse_core_info()
NC, NS, LANES = _info.num_cores, _info.num_subcores, _info.num_lanes
NW = NC * NS
N_TOK = B * L
PER_W = N_TOK // NW
SEQ_PER_W = PER_W // L
NV = D // 16
C0 = 120
C1 = L - C0


def _rsqrt16(v):
    i = plsc.bitcast(v, jnp.int32)
    magic = jnp.full((16,), 0x5F3759DF, jnp.int32)
    one = jnp.full((16,), 1, jnp.int32)
    y = plsc.bitcast(magic - lax.shift_right_logical(i, one), jnp.float32)
    half = v * 0.5
    y = y * (1.5 - half * y * y)
    y = y * (1.5 - half * y * y)
    return y


def _make_kernel():
    mesh = plsc.VectorSubcoreMesh(core_axis_name="c", subcore_axis_name="s")

    @functools.partial(
        pl.kernel,
        mesh=mesh,
        out_type=jax.ShapeDtypeStruct((N_TOK, D), jnp.float32),
        compiler_params=pltpu.CompilerParams(
            needs_layout_passes=False, use_tc_tiling_on_sc=False),
        scratch_types=[
            pltpu.VMEM((PER_W,), jnp.int32),
            pltpu.VMEM((L, D), jnp.float32),
            pltpu.VMEM((L, D), jnp.float32),
            pltpu.VMEM((L, D), jnp.float32),
            pltpu.VMEM((L, D), jnp.float32),
            pltpu.VMEM((L, D), jnp.float32),
            pltpu.VMEM((D,), jnp.float32),
            pltpu.VMEM((D,), jnp.float32),
            pltpu.SemaphoreType.DMA,
            pltpu.SemaphoreType.DMA,
            pltpu.SemaphoreType.DMA,
            pltpu.SemaphoreType.DMA,
            pltpu.SemaphoreType.DMA,
            pltpu.SemaphoreType.DMA,
        ],
    )
    def emb_kernel(ids_hbm, w_hbm, pe_hbm, g_hbm, b_hbm, out_hbm,
                   ids_v, pe_v, rin0, rin1, rout0, rout1, g_v, b_v,
                   gsem0, gsem1, ssem0, ssem1, dsem0, dsem1):
        cid = lax.axis_index("c")
        sid = lax.axis_index("s")
        wid = sid * NC + cid
        base = wid * PER_W

        pltpu.sync_copy(ids_hbm.at[pl.ds(base, PER_W)], ids_v)
        pltpu.sync_copy(pe_hbm, pe_v)
        pltpu.sync_copy(g_hbm, g_v)
        pltpu.sync_copy(b_hbm, b_v)

        rins = (rin0, rin1)
        routs = (rout0, rout1)
        gsems = (gsem0, gsem1)
        ssems = (ssem0, ssem1)
        dsems = (dsem0, dsem1)
        gvs = [g_v[pl.ds(16 * i, 16)] for i in range(NV)]
        bvs = [b_v[pl.ds(16 * i, 16)] for i in range(NV)]
        inv_d = 1.0 / D

        def start_gather(g, bank):
            off = g * L
            pltpu.async_copy(
                w_hbm.at[ids_v.at[pl.ds(off, C0)]],
                rins[bank].at[pl.ds(0, C0)], gsems[bank])
            rin = rins[bank]

            def dma_chunk(k, carry):
                idv = ids_v[pl.ds(off + C0 + 16 * k, 16)]
                for j in range(16):
                    pltpu.async_copy(
                        w_hbm.at[idv[j]], rin.at[C0 + 16 * k + j], dsems[bank])
                return carry

            lax.fori_loop(0, C1 // 16, dma_chunk, 0)

        def wait_gather(bank):
            pltpu.make_async_copy(
                w_hbm.at[ids_v.at[pl.ds(0, C0)]],
                rins[bank].at[pl.ds(0, C0)], gsems[bank]).wait()
            pltpu.make_async_copy(
                w_hbm.at[pl.ds(0, C1)],
                rins[bank].at[pl.ds(C0, C1)], dsems[bank]).wait()

        def start_scatter(g, bank):
            pltpu.async_copy(
                routs[bank], out_hbm.at[pl.ds(base + g * L, L)], ssems[bank])

        def wait_scatter(bank):
            pltpu.make_async_copy(
                routs[bank], out_hbm.at[pl.ds(base, L)], ssems[bank]).wait()

        def compute_row(rin, rout, r):
            x = [rin[r, pl.ds(16 * i, 16)] + pe_v[r, pl.ds(16 * i, 16)]
                 for i in range(NV)]
            tot = jnp.sum((x[0] + x[1]) + (x[2] + x[3]))
            ss = jnp.sum((x[0] * x[0] + x[1] * x[1])
                         + (x[2] * x[2] + x[3] * x[3]))
            mean = tot * inv_d
            var = ss * inv_d - mean * mean
            rv = _rsqrt16(jnp.broadcast_to(var + EPS, (16,)))
            for i in range(NV):
                rout[r, pl.ds(16 * i, 16)] = (x[i] - mean) * rv * gvs[i] + bvs[i]

        def compute_seq(bank):
            rin = rins[bank]
            rout = routs[bank]

            def row_body(rr, c):
                compute_row(rin, rout, 2 * rr)
                compute_row(rin, rout, 2 * rr + 1)
                return c

            lax.fori_loop(0, L // 2, row_body, 0)

        start_gather(0, 0)
        start_gather(1, 1)

        def step(gg, carry):
            for bank in range(2):
                g = gg * 2 + bank
                wait_gather(bank)

                @pl.when(g >= 2)
                def _():
                    wait_scatter(bank)

                compute_seq(bank)
                start_scatter(g, bank)

                @pl.when(g + 2 < SEQ_PER_W)
                def _():
                    start_gather(g + 2, bank)
            return carry

        lax.fori_loop(0, SEQ_PER_W // 2, step, 0)

        wait_scatter(0)
        wait_scatter(1)

    return emb_kernel


_emb_kernel = _make_kernel()


@jax.jit
def kernel(input_ids, W_emb, pe, ln_gamma, ln_beta):
    ids_flat = input_ids.reshape(-1)
    pe_l = pe[:L]
    out = _emb_kernel(ids_flat, W_emb, pe_l, ln_gamma, ln_beta)
    return out.reshape(B, L, D)


# speedup vs baseline: 1.0167x; 1.0167x over previous
import functools

import jax
import jax.numpy as jnp
from jax import lax
from jax.experimental import pallas as pl
from jax.experimental.pallas import tpu as pltpu
from jax.experimental.pallas import tpu_sc as plsc

B = 4096
L = 200
D = 64
EPS = 1e-12

_info = plsc.get_sparse_core_info()
NC, NS, LANES = _info.num_cores, _info.num_subcores, _info.num_lanes
NW = NC * NS
N_TOK = B * L
PER_W = N_TOK // NW
SEQ_PER_W = PER_W // L
NV = D // 16
C0 = 128
C1 = L - C0


def _rsqrt16(v):
    i = plsc.bitcast(v, jnp.int32)
    magic = jnp.full((16,), 0x5F3759DF, jnp.int32)
    one = jnp.full((16,), 1, jnp.int32)
    y = plsc.bitcast(magic - lax.shift_right_logical(i, one), jnp.float32)
    half = v * 0.5
    y = y * (1.5 - half * y * y)
    y = y * (1.5 - half * y * y)
    return y


def _make_kernel():
    mesh = plsc.VectorSubcoreMesh(core_axis_name="c", subcore_axis_name="s")

    @functools.partial(
        pl.kernel,
        mesh=mesh,
        out_type=jax.ShapeDtypeStruct((N_TOK, D), jnp.float32),
        compiler_params=pltpu.CompilerParams(
            needs_layout_passes=False, use_tc_tiling_on_sc=False),
        scratch_types=[
            pltpu.VMEM((PER_W,), jnp.int32),
            pltpu.VMEM((L, D), jnp.float32),
            pltpu.VMEM((L, D), jnp.float32),
            pltpu.VMEM((L, D), jnp.float32),
            pltpu.VMEM((L, D), jnp.float32),
            pltpu.VMEM((L, D), jnp.float32),
            pltpu.VMEM((D,), jnp.float32),
            pltpu.VMEM((D,), jnp.float32),
            pltpu.SemaphoreType.DMA,
            pltpu.SemaphoreType.DMA,
            pltpu.SemaphoreType.DMA,
            pltpu.SemaphoreType.DMA,
        ],
    )
    def emb_kernel(ids_hbm, w_hbm, pe_hbm, g_hbm, b_hbm, out_hbm,
                   ids_v, pe_v, rin0, rin1, rout0, rout1, g_v, b_v,
                   gsem0, gsem1, ssem0, ssem1):
        cid = lax.axis_index("c")
        sid = lax.axis_index("s")
        wid = sid * NC + cid
        base = wid * PER_W

        pltpu.sync_copy(ids_hbm.at[pl.ds(base, PER_W)], ids_v)
        pltpu.sync_copy(pe_hbm, pe_v)
        pltpu.sync_copy(g_hbm, g_v)
        pltpu.sync_copy(b_hbm, b_v)

        rins = (rin0, rin1)
        routs = (rout0, rout1)
        gsems = (gsem0, gsem1)
        ssems = (ssem0, ssem1)
        gvs = [g_v[pl.ds(16 * i, 16)] for i in range(NV)]
        bvs = [b_v[pl.ds(16 * i, 16)] for i in range(NV)]
        inv_d = 1.0 / D

        def start_gather(g, bank):
            off = g * L
            pltpu.async_copy(
                w_hbm.at[ids_v.at[pl.ds(off, C0)]],
                rins[bank].at[pl.ds(0, C0)], gsems[bank])
            pltpu.async_copy(
                w_hbm.at[ids_v.at[pl.ds(off + C0, C1)]],
                rins[bank].at[pl.ds(C0, C1)], gsems[bank])

        def wait_gather(bank):
            pltpu.make_async_copy(
                w_hbm.at[ids_v.at[pl.ds(0, C0)]],
                rins[bank].at[pl.ds(0, C0)], gsems[bank]).wait()
            pltpu.make_async_copy(
                w_hbm.at[ids_v.at[pl.ds(C0, C1)]],
                rins[bank].at[pl.ds(C0, C1)], gsems[bank]).wait()

        def start_scatter(g, bank):
            pltpu.async_copy(
                routs[bank], out_hbm.at[pl.ds(base + g * L, L)], ssems[bank])

        def wait_scatter(bank):
            pltpu.make_async_copy(
                routs[bank], out_hbm.at[pl.ds(base, L)], ssems[bank]).wait()

        def compute_row(rin, rout, r):
            x = [rin[r, pl.ds(16 * i, 16)] + pe_v[r, pl.ds(16 * i, 16)]
                 for i in range(NV)]
            tot = jnp.sum((x[0] + x[1]) + (x[2] + x[3]))
            ss = jnp.sum((x[0] * x[0] + x[1] * x[1])
                         + (x[2] * x[2] + x[3] * x[3]))
            mean = tot * inv_d
            var = ss * inv_d - mean * mean
            rv = _rsqrt16(jnp.broadcast_to(var + EPS, (16,)))
            for i in range(NV):
                rout[r, pl.ds(16 * i, 16)] = (x[i] - mean) * rv * gvs[i] + bvs[i]

        def compute_seq(bank):
            rin = rins[bank]
            rout = routs[bank]

            def row_body(rr, c):
                compute_row(rin, rout, 2 * rr)
                compute_row(rin, rout, 2 * rr + 1)
                return c

            lax.fori_loop(0, L // 2, row_body, 0)

        start_gather(0, 0)
        start_gather(1, 1)

        def step(gg, carry):
            for bank in range(2):
                g = gg * 2 + bank
                wait_gather(bank)

                @pl.when(g >= 2)
                def _():
                    wait_scatter(bank)

                compute_seq(bank)
                start_scatter(g, bank)

                @pl.when(g + 2 < SEQ_PER_W)
                def _():
                    start_gather(g + 2, bank)
            return carry

        lax.fori_loop(0, SEQ_PER_W // 2, step, 0)

        wait_scatter(0)
        wait_scatter(1)

    return emb_kernel


_emb_kernel = _make_kernel()


@jax.jit
def kernel(input_ids, W_emb, pe, ln_gamma, ln_beta):
    ids_flat = input_ids.reshape(-1)
    pe_l = pe[:L]
    out = _emb_kernel(ids_flat, W_emb, pe_l, ln_gamma, ln_beta)
    return out.reshape(B, L, D)
